# Initial kernel scaffold; baseline (speedup 1.0000x reference)
import jax
import jax.numpy as jnp
from jax.experimental import pallas as pl


def kernel(state, x, edge_index, g1_Wa, g1_ba, g1_Wb, g1_bb, g2_Wa, g2_ba, g2_Wb, g2_bb, fc1_W, fc1_b, fc2_W, fc2_b, mean_W, mean_b, ls_W, ls_b):
    raise NotImplementedError("write your pallas kernel here")


# speedup vs baseline: 7.2368x; 7.2368x over previous
import functools

import jax
import jax.numpy as jnp
from jax import lax
from jax.experimental import pallas as pl
from jax.experimental.pallas import tpu as pltpu
from jax.experimental.pallas import tpu_sc as plsc

_NC = 2
_NS = 16
_NW = _NC * _NS
_LANES = 16
_CNTW = 16

_SC_PARAMS = pltpu.CompilerParams(use_tc_tiling_on_sc=False,
                                  needs_layout_passes=False)


def _pick_chunk(epw: int) -> int:
    for c in range(128, 7, -8):
        if epw % c == 0:
            return c
    raise ValueError(f"no valid chunk for {epw} edges per tile")


def _pick_zr(rpt: int) -> int:
    for c in range(32, 0, -1):
        if rpt % c == 0:
            return c
    return 1


def _make_sc_edge_pass(n_nodes: int, d: int, n_edges: int, col_split: bool,
                       with_cnt: bool):
    nw = _NS if col_split else _NW
    assert n_edges % nw == 0
    epw = n_edges // nw
    chunk = _pick_chunk(epw)
    nch = epw // chunk
    sb = 25
    while nch % sb:
        sb -= 1
    nsb = nch // sb
    assert n_nodes % _NS == 0
    rpt = n_nodes // _NS
    zr = _pick_zr(rpt)
    nz = rpt // zr

    mesh = plsc.VectorSubcoreMesh(
        core_axis_name="c", subcore_axis_name="s",
        num_cores=_NC, num_subcores=_NS)

    out_type = [jax.ShapeDtypeStruct((_NC, n_nodes, d), jnp.float32)]
    scratch = [
        pltpu.VMEM((sb, chunk), jnp.int32),
        pltpu.VMEM((sb, chunk), jnp.int32),
        pltpu.VMEM((chunk, d), jnp.float32),
        pltpu.VMEM((chunk, d), jnp.float32),
        pltpu.VMEM((zr, d), jnp.float32),
        pltpu.VMEM_SHARED((n_nodes, d), jnp.float32),
        pltpu.SemaphoreType.DMA,
        pltpu.SemaphoreType.DMA,
    ]
    if with_cnt:
        out_type.append(jax.ShapeDtypeStruct((_NC, _NS, n_nodes), jnp.float32))
        scratch.append(pltpu.VMEM((n_nodes,), jnp.float32))

    def body(*refs):
        n_tbl = 4 if col_split else 2
        tables = refs[:n_tbl]
        eidx_hbm, s_out = refs[n_tbl:n_tbl + 2]
        rest = refs[n_tbl + 2:]
        if with_cnt:
            c_out, rest = rest[0], rest[1:]
        didx, sidx, buf_a, buf_b, zbuf, acc_sh, sem_a, sem_b = rest[:8]
        cnt_v = rest[8] if with_cnt else None

        cid = lax.axis_index("c")
        sid = lax.axis_index("s")

        def zero_rows(r, _):
            for k in range(d // _LANES):
                zbuf[r, pl.ds(k * _LANES, _LANES)] = jnp.zeros((_LANES,), jnp.float32)
            return 0
        lax.fori_loop(0, zr, zero_rows, 0)
        for t in range(nz):
            pltpu.sync_copy(zbuf, acc_sh.at[pl.ds(sid * rpt + t * zr, zr)])
        if with_cnt:
            def zero_cnt(i, _):
                cnt_v[pl.ds(i * _LANES, _LANES)] = jnp.zeros((_LANES,), jnp.float32)
                return 0
            lax.fori_loop(0, n_nodes // _LANES, zero_cnt, 0)
        plsc.subcore_barrier()

        def gather(pi_hbm, pj_hbm, di, si):
            cp_a = pltpu.async_copy(pi_hbm.at[di], buf_a, sem_a)
            cp_b = pltpu.async_copy(pj_hbm.at[si], buf_b, sem_b)
            cp_a.wait()
            cp_b.wait()

        def chunk_body(k, _):
            di = didx.at[k]
            si = sidx.at[k]
            if col_split:
                @pl.when(cid == 0)
                def _():
                    gather(tables[0], tables[2], di, si)

                @pl.when(cid != 0)
                def _():
                    gather(tables[1], tables[3], di, si)
            else:
                gather(tables[0], tables[1], di, si)

            def row(r, _2):
                for kk in range(d // _LANES):
                    sl = pl.ds(kk * _LANES, _LANES)
                    buf_a[r, sl] = jnp.maximum(buf_a[r, sl] + buf_b[r, sl], 0.0)
                return 0
            lax.fori_loop(0, chunk, row, 0)

            pltpu.sync_copy(buf_a, acc_sh.at[di], add=True)
            if with_cnt:
                for j in range(chunk // _LANES):
                    idx16 = didx[k, pl.ds(j * _LANES, _LANES)]
                    plsc.addupdate_scatter(
                        cnt_v, [idx16], jnp.ones((_LANES,), jnp.float32))
            return 0

        def super_body(s, _):
            if col_split:
                pltpu.sync_copy(eidx_hbm.at[1, sid, pl.ds(s * sb, sb)], didx)
                pltpu.sync_copy(eidx_hbm.at[0, sid, pl.ds(s * sb, sb)], sidx)
            else:
                pltpu.sync_copy(eidx_hbm.at[1, cid, sid, pl.ds(s * sb, sb)], didx)
                pltpu.sync_copy(eidx_hbm.at[0, cid, sid, pl.ds(s * sb, sb)], sidx)
            lax.fori_loop(0, sb, chunk_body, 0)
            return 0
        lax.fori_loop(0, nsb, super_body, 0)

        plsc.subcore_barrier()

        pltpu.sync_copy(acc_sh.at[pl.ds(sid * rpt, rpt)],
                        s_out.at[cid, pl.ds(sid * rpt, rpt)])
        if with_cnt:
            pltpu.sync_copy(cnt_v, c_out.at[cid, sid])

    return pl.kernel(body, out_type=out_type, mesh=mesh, scratch_types=scratch,
                     compiler_params=_SC_PARAMS)


def _proj_body(x_ref, wi_ref, wj_ref, ba_ref, pi_ref, pj_ref):
    xb = x_ref[...]
    pi_ref[...] = (jnp.dot(xb, wi_ref[...], preferred_element_type=jnp.float32)
                   + ba_ref[...])
    pj_ref[...] = jnp.dot(xb, wj_ref[...], preferred_element_type=jnp.float32)


def _cnt_column(c_ref):
    c2 = c_ref[0] + c_ref[1]
    ones = jnp.ones((_NS, 1), jnp.float32)
    return lax.dot_general(c2, ones, (((0,), (0,)), ((), ())),
                           preferred_element_type=jnp.float32)


def _mid_body(s_ref, c_ref, wb_ref, bb_ref,
              wil_ref, wih_ref, wjl_ref, wjh_ref, bil_ref, bih_ref,
              qil_ref, qih_ref, qjl_ref, qjh_ref):
    s = s_ref[0] + s_ref[1]
    cnt = _cnt_column(c_ref)
    inv = 1.0 / jnp.maximum(cnt, 1.0)
    ind = (cnt > 0.0).astype(jnp.float32)
    h = jnp.maximum(
        jnp.dot(s * inv, wb_ref[...], preferred_element_type=jnp.float32)
        + ind * bb_ref[...], 0.0)
    qil_ref[...] = (jnp.dot(h, wil_ref[...], preferred_element_type=jnp.float32)
                    + bil_ref[...])
    qih_ref[...] = (jnp.dot(h, wih_ref[...], preferred_element_type=jnp.float32)
                    + bih_ref[...])
    qjl_ref[...] = jnp.dot(h, wjl_ref[...], preferred_element_type=jnp.float32)
    qjh_ref[...] = jnp.dot(h, wjh_ref[...], preferred_element_type=jnp.float32)


def _final_body(n_nodes, s_ref, c_ref, st_ref, wb2l_ref, wb2h_ref, bb2_ref,
                f1s_ref, f1g_ref, b1_ref, f2_ref, b2_ref,
                wm_ref, bm_ref, wl_ref, bl_ref,
                mean_ref, ls_ref):
    cnt = _cnt_column(c_ref)
    inv = 1.0 / jnp.maximum(cnt, 1.0)
    ind = (cnt > 0.0).astype(jnp.float32)
    scale = 1.0 / n_nodes
    t_lo = jnp.sum(s_ref[0] * inv, axis=0, keepdims=True) * scale
    t_hi = jnp.sum(s_ref[1] * inv, axis=0, keepdims=True) * scale
    frac = jnp.sum(ind) * scale
    ge = (jnp.dot(t_lo, wb2l_ref[...], preferred_element_type=jnp.float32)
          + jnp.dot(t_hi, wb2h_ref[...], preferred_element_type=jnp.float32)
          + frac * bb2_ref[...])
    z = jnp.maximum(
        jnp.dot(st_ref[...], f1s_ref[...], preferred_element_type=jnp.float32)
        + jnp.dot(ge, f1g_ref[...], preferred_element_type=jnp.float32)
        + b1_ref[...], 0.0)
    z = jnp.maximum(
        jnp.dot(z, f2_ref[...], preferred_element_type=jnp.float32) + b2_ref[...], 0.0)
    mean_ref[...] = (jnp.dot(z, wm_ref[...], preferred_element_type=jnp.float32)
                     + bm_ref[...])
    ls_ref[...] = jnp.clip(
        jnp.dot(z, wl_ref[...], preferred_element_type=jnp.float32) + bl_ref[...],
        -20.0, 2.0)


def kernel(state, x, edge_index, g1_Wa, g1_ba, g1_Wb, g1_bb,
           g2_Wa, g2_ba, g2_Wb, g2_bb, fc1_W, fc1_b, fc2_W, fc2_b,
           mean_W, mean_b, ls_W, ls_b):
    n, f = x.shape
    hh = g1_Wa.shape[0]
    g = g2_Wa.shape[0]
    e = edge_index.shape[1]
    rb = 1000
    assert n % rb == 0

    ei32 = edge_index.astype(jnp.int32)
    epw1 = e // _NW
    chunk1 = _pick_chunk(epw1)
    eidx1 = ei32.reshape(2, _NC, _NS, epw1 // chunk1, chunk1)
    epw2 = e // _NS
    chunk2 = _pick_chunk(epw2)
    eidx2 = ei32.reshape(2, _NS, epw2 // chunk2, chunk2)
    g2 = g // 2

    wi_t = g1_Wa[:, :f].T
    wj_t = g1_Wa[:, f:].T
    wb1_t = g1_Wb.T
    wa2i_t = g2_Wa[:, :hh].T
    wa2j_t = g2_Wa[:, hh:].T
    wb2_t = g2_Wb.T
    fc1_t, fc2_t = fc1_W.T, fc2_W.T
    mean_t, ls_t = mean_W.T, ls_W.T
    ba1 = g1_ba.reshape(1, hh)
    bb1 = g1_bb.reshape(1, hh)
    ba2 = g2_ba.reshape(1, g)
    bb2 = g2_bb.reshape(1, g)
    b1 = fc1_b.reshape(1, -1)
    b2 = fc2_b.reshape(1, -1)
    bm = mean_b.reshape(1, -1)
    bl = ls_b.reshape(1, -1)

    pi, pj = pl.pallas_call(
        _proj_body,
        out_shape=[
            jax.ShapeDtypeStruct((n, hh), jnp.float32),
            jax.ShapeDtypeStruct((n, hh), jnp.float32),
        ],
    )(x, wi_t, wj_t, ba1)

    s1p, cntp = _make_sc_edge_pass(n, hh, e, col_split=False, with_cnt=True)(
        pi, pj, eidx1)

    qshape = jax.ShapeDtypeStruct((n, g2), jnp.float32)
    qi_lo, qi_hi, qj_lo, qj_hi = pl.pallas_call(
        _mid_body,
        out_shape=[qshape, qshape, qshape, qshape],
    )(s1p, cntp, wb1_t, bb1,
      wa2i_t[:, :g2], wa2i_t[:, g2:], wa2j_t[:, :g2], wa2j_t[:, g2:],
      ba2[:, :g2], ba2[:, g2:])

    (s2p,) = _make_sc_edge_pass(n, g2, e, col_split=True, with_cnt=False)(
        qi_lo, qi_hi, qj_lo, qj_hi, eidx2)

    mean, log_std = pl.pallas_call(
        functools.partial(_final_body, n),
        out_shape=[
            jax.ShapeDtypeStruct((1, mean_t.shape[1]), jnp.float32),
            jax.ShapeDtypeStruct((1, ls_t.shape[1]), jnp.float32),
        ],
    )(s2p, cntp, state, wb2_t[:g2], wb2_t[g2:], bb2,
      fc1_t[:f], fc1_t[f:], b1, fc2_t, b2, mean_t, bm, ls_t, bl)

    return (mean, log_std)


# speedup vs baseline: 12.0687x; 1.6677x over previous
import functools

import jax
import jax.numpy as jnp
from jax import lax
from jax.experimental import pallas as pl
from jax.experimental.pallas import tpu as pltpu
from jax.experimental.pallas import tpu_sc as plsc

_NC = 2
_NS = 16
_LANES = 16

_SC_PARAMS = pltpu.CompilerParams(use_tc_tiling_on_sc=False,
                                  needs_layout_passes=False)


def _pick_chunk(epw: int) -> int:
    for c in range(128, 7, -8):
        if epw % c == 0 and (epw // c) % 2 == 0:
            return c
    raise ValueError(f"no valid chunk for {epw} edges per tile")


def _pick_sb(nch: int) -> int:
    for c in range(64, 1, -2):
        if nch % c == 0:
            return c
    return 2


def _pick_zr(rpt: int) -> int:
    for c in range(32, 0, -1):
        if rpt % c == 0:
            return c
    return 1


def _make_sc_edge_pass(n_nodes: int, d: int, n_edges: int, with_cnt: bool):
    assert n_edges % _NS == 0
    epw = n_edges // _NS
    chunk = _pick_chunk(epw)
    nch = epw // chunk
    sb = _pick_sb(nch)
    nsb = nch // sb
    assert n_nodes % _NS == 0
    rpt = n_nodes // _NS
    zr = _pick_zr(rpt)
    nz = rpt // zr

    mesh = plsc.VectorSubcoreMesh(
        core_axis_name="c", subcore_axis_name="s",
        num_cores=_NC, num_subcores=_NS)

    out_type = [jax.ShapeDtypeStruct((_NC, n_nodes, d), jnp.float32)]
    scratch = [
        pltpu.VMEM((sb, chunk), jnp.int32),
        pltpu.VMEM((sb, chunk), jnp.int32),
        pltpu.VMEM((chunk, d), jnp.float32),
        pltpu.VMEM((chunk, d), jnp.float32),
        pltpu.VMEM((chunk, d), jnp.float32),
        pltpu.VMEM((chunk, d), jnp.float32),
        pltpu.VMEM((chunk, d), jnp.float32),
        pltpu.VMEM((chunk, d), jnp.float32),
        pltpu.VMEM((zr, d), jnp.float32),
        pltpu.VMEM_SHARED((n_nodes, d), jnp.float32),
        pltpu.SemaphoreType.DMA,
        pltpu.SemaphoreType.DMA,
        pltpu.SemaphoreType.DMA,
        pltpu.SemaphoreType.DMA,
        pltpu.SemaphoreType.DMA,
        pltpu.SemaphoreType.DMA,
    ]
    if with_cnt:
        out_type.append(jax.ShapeDtypeStruct((_NS, n_nodes), jnp.float32))
        scratch.append(pltpu.VMEM((n_nodes,), jnp.float32))

    def body(*refs):
        tables = refs[:4]
        eidx_hbm, s_out = refs[4:6]
        rest = refs[6:]
        if with_cnt:
            c_out, rest = rest[0], rest[1:]
        (didx, sidx, a0, a1, b0, b1, c0, c1, zbuf, acc_sh,
         sga0, sga1, sgb0, sgb1, ss0, ss1) = rest[:16]
        cnt_v = rest[16] if with_cnt else None
        abuf = (a0, a1)
        bbuf = (b0, b1)
        cbuf = (c0, c1)
        sga = (sga0, sga1)
        sgb = (sgb0, sgb1)
        ss = (ss0, ss1)

        cid = lax.axis_index("c")
        sid = lax.axis_index("s")

        def zero_rows(r, _):
            for k in range(d // _LANES):
                zbuf[r, pl.ds(k * _LANES, _LANES)] = jnp.zeros((_LANES,), jnp.float32)
            return 0
        lax.fori_loop(0, zr, zero_rows, 0)
        for t in range(nz):
            pltpu.sync_copy(zbuf, acc_sh.at[pl.ds(sid * rpt + t * zr, zr)])
        if with_cnt:
            def zero_cnt(i, _):
                cnt_v[pl.ds(i * _LANES, _LANES)] = jnp.zeros((_LANES,), jnp.float32)
                return 0
            lax.fori_loop(0, n_nodes // _LANES, zero_cnt, 0)
        plsc.subcore_barrier()

        def issue_gathers(c, slot):
            di = didx.at[c]
            si = sidx.at[c]

            @pl.when(cid == 0)
            def _():
                pltpu.async_copy(tables[0].at[di], abuf[slot], sga[slot])
                pltpu.async_copy(tables[2].at[si], bbuf[slot], sgb[slot])

            @pl.when(cid != 0)
            def _():
                pltpu.async_copy(tables[1].at[di], abuf[slot], sga[slot])
                pltpu.async_copy(tables[3].at[si], bbuf[slot], sgb[slot])

        def wait_gathers(slot):
            pltpu.make_async_copy(tables[0].at[pl.ds(0, chunk)],
                                  abuf[slot], sga[slot]).wait()
            pltpu.make_async_copy(tables[2].at[pl.ds(0, chunk)],
                                  bbuf[slot], sgb[slot]).wait()

        def wait_scatter(slot):
            pltpu.make_async_copy(cbuf[slot], acc_sh.at[pl.ds(0, chunk)],
                                  ss[slot]).wait()

        def super_body(s, _):
            pltpu.sync_copy(eidx_hbm.at[1, sid, pl.ds(s * sb, sb)], didx)
            pltpu.sync_copy(eidx_hbm.at[0, sid, pl.ds(s * sb, sb)], sidx)
            issue_gathers(0, 0)

            def pair_body(t, _2):
                for b in (0, 1):
                    c = 2 * t + b
                    if b == 0:
                        issue_gathers(c + 1, 1)
                    else:
                        @pl.when(t < sb // 2 - 1)
                        def _():
                            issue_gathers(c + 1, 0)
                    wait_gathers(b)
                    @pl.when((s > 0) | (t > 0))
                    def _():
                        wait_scatter(b)
                    def row(r, _3):
                        for k in range(d // _LANES):
                            sl = pl.ds(k * _LANES, _LANES)
                            cbuf[b][r, sl] = jnp.maximum(
                                abuf[b][r, sl] + bbuf[b][r, sl], 0.0)
                        return 0
                    lax.fori_loop(0, chunk, row, 0)
                    pltpu.async_copy(cbuf[b], acc_sh.at[didx.at[c]], ss[b],
                                     add=True)
                    if with_cnt:
                        @pl.when(cid == 0)
                        def _():
                            for j in range(chunk // _LANES):
                                idx16 = didx[c, pl.ds(j * _LANES, _LANES)]
                                plsc.addupdate_scatter(
                                    cnt_v, [idx16],
                                    jnp.ones((_LANES,), jnp.float32))
                return 0
            lax.fori_loop(0, sb // 2, pair_body, 0)
            return 0
        lax.fori_loop(0, nsb, super_body, 0)

        wait_scatter(0)
        wait_scatter(1)
        plsc.subcore_barrier()

        pltpu.sync_copy(acc_sh.at[pl.ds(sid * rpt, rpt)],
                        s_out.at[cid, pl.ds(sid * rpt, rpt)])
        if with_cnt:
            @pl.when(cid == 0)
            def _():
                pltpu.sync_copy(cnt_v, c_out.at[sid])

    return pl.kernel(body, out_type=out_type, mesh=mesh, scratch_types=scratch,
                     compiler_params=_SC_PARAMS)


def _cnt_column(c_ref):
    ones = jnp.ones((_NS, 1), jnp.float32)
    return lax.dot_general(c_ref[...], ones, (((0,), (0,)), ((), ())),
                           preferred_element_type=jnp.float32)


def _proj_body(x_ref, wil_ref, wih_ref, wjl_ref, wjh_ref, bal_ref, bah_ref,
               pil_ref, pih_ref, pjl_ref, pjh_ref):
    xb = x_ref[...]
    pil_ref[...] = (jnp.dot(xb, wil_ref[...], preferred_element_type=jnp.float32)
                    + bal_ref[...])
    pih_ref[...] = (jnp.dot(xb, wih_ref[...], preferred_element_type=jnp.float32)
                    + bah_ref[...])
    pjl_ref[...] = jnp.dot(xb, wjl_ref[...], preferred_element_type=jnp.float32)
    pjh_ref[...] = jnp.dot(xb, wjh_ref[...], preferred_element_type=jnp.float32)


def _mid_body(s_ref, c_ref, wbl_ref, wbh_ref, bb_ref,
              wil_ref, wih_ref, wjl_ref, wjh_ref, bil_ref, bih_ref,
              qil_ref, qih_ref, qjl_ref, qjh_ref):
    cnt = _cnt_column(c_ref)
    inv = 1.0 / jnp.maximum(cnt, 1.0)
    ind = (cnt > 0.0).astype(jnp.float32)
    h = jnp.maximum(
        jnp.dot(s_ref[0] * inv, wbl_ref[...], preferred_element_type=jnp.float32)
        + jnp.dot(s_ref[1] * inv, wbh_ref[...], preferred_element_type=jnp.float32)
        + ind * bb_ref[...], 0.0)
    qil_ref[...] = (jnp.dot(h, wil_ref[...], preferred_element_type=jnp.float32)
                    + bil_ref[...])
    qih_ref[...] = (jnp.dot(h, wih_ref[...], preferred_element_type=jnp.float32)
                    + bih_ref[...])
    qjl_ref[...] = jnp.dot(h, wjl_ref[...], preferred_element_type=jnp.float32)
    qjh_ref[...] = jnp.dot(h, wjh_ref[...], preferred_element_type=jnp.float32)


def _final_body(n_nodes, s_ref, c_ref, st_ref, wb2l_ref, wb2h_ref, bb2_ref,
                f1s_ref, f1g_ref, b1_ref, f2_ref, b2_ref,
                wm_ref, bm_ref, wl_ref, bl_ref,
                mean_ref, ls_ref):
    cnt = _cnt_column(c_ref)
    inv = 1.0 / jnp.maximum(cnt, 1.0)
    ind = (cnt > 0.0).astype(jnp.float32)
    scale = 1.0 / n_nodes
    t_lo = jnp.sum(s_ref[0] * inv, axis=0, keepdims=True) * scale
    t_hi = jnp.sum(s_ref[1] * inv, axis=0, keepdims=True) * scale
    frac = jnp.sum(ind) * scale
    ge = (jnp.dot(t_lo, wb2l_ref[...], preferred_element_type=jnp.float32)
          + jnp.dot(t_hi, wb2h_ref[...], preferred_element_type=jnp.float32)
          + frac * bb2_ref[...])
    z = jnp.maximum(
        jnp.dot(st_ref[...], f1s_ref[...], preferred_element_type=jnp.float32)
        + jnp.dot(ge, f1g_ref[...], preferred_element_type=jnp.float32)
        + b1_ref[...], 0.0)
    z = jnp.maximum(
        jnp.dot(z, f2_ref[...], preferred_element_type=jnp.float32) + b2_ref[...], 0.0)
    mean_ref[...] = (jnp.dot(z, wm_ref[...], preferred_element_type=jnp.float32)
                     + bm_ref[...])
    ls_ref[...] = jnp.clip(
        jnp.dot(z, wl_ref[...], preferred_element_type=jnp.float32) + bl_ref[...],
        -20.0, 2.0)


def kernel(state, x, edge_index, g1_Wa, g1_ba, g1_Wb, g1_bb,
           g2_Wa, g2_ba, g2_Wb, g2_bb, fc1_W, fc1_b, fc2_W, fc2_b,
           mean_W, mean_b, ls_W, ls_b):
    n, f = x.shape
    hh = g1_Wa.shape[0]
    g = g2_Wa.shape[0]
    e = edge_index.shape[1]

    epw = e // _NS
    chunk = _pick_chunk(epw)
    nch = epw // chunk
    eidx = edge_index.astype(jnp.int32).reshape(2, _NS, nch, chunk)
    hh2 = hh // 2
    g2c = g // 2

    wi_t = g1_Wa[:, :f].T
    wj_t = g1_Wa[:, f:].T
    wb1_t = g1_Wb.T
    wa2i_t = g2_Wa[:, :hh].T
    wa2j_t = g2_Wa[:, hh:].T
    wb2_t = g2_Wb.T
    fc1_t, fc2_t = fc1_W.T, fc2_W.T
    mean_t, ls_t = mean_W.T, ls_W.T
    ba1 = g1_ba.reshape(1, hh)
    bb1 = g1_bb.reshape(1, hh)
    ba2 = g2_ba.reshape(1, g)
    bb2 = g2_bb.reshape(1, g)
    b1 = fc1_b.reshape(1, -1)
    b2 = fc2_b.reshape(1, -1)
    bm = mean_b.reshape(1, -1)
    bl = ls_b.reshape(1, -1)

    pshape = jax.ShapeDtypeStruct((n, hh2), jnp.float32)
    pi_lo, pi_hi, pj_lo, pj_hi = pl.pallas_call(
        _proj_body,
        out_shape=[pshape, pshape, pshape, pshape],
    )(x, wi_t[:, :hh2], wi_t[:, hh2:], wj_t[:, :hh2], wj_t[:, hh2:],
      ba1[:, :hh2], ba1[:, hh2:])

    s1p, cntp = _make_sc_edge_pass(n, hh2, e, with_cnt=True)(
        pi_lo, pi_hi, pj_lo, pj_hi, eidx)

    qshape = jax.ShapeDtypeStruct((n, g2c), jnp.float32)
    qi_lo, qi_hi, qj_lo, qj_hi = pl.pallas_call(
        _mid_body,
        out_shape=[qshape, qshape, qshape, qshape],
    )(s1p, cntp, wb1_t[:hh2], wb1_t[hh2:], bb1,
      wa2i_t[:, :g2c], wa2i_t[:, g2c:], wa2j_t[:, :g2c], wa2j_t[:, g2c:],
      ba2[:, :g2c], ba2[:, g2c:])

    (s2p,) = _make_sc_edge_pass(n, g2c, e, with_cnt=False)(
        qi_lo, qi_hi, qj_lo, qj_hi, eidx)

    mean, log_std = pl.pallas_call(
        functools.partial(_final_body, n),
        out_shape=[
            jax.ShapeDtypeStruct((1, mean_t.shape[1]), jnp.float32),
            jax.ShapeDtypeStruct((1, ls_t.shape[1]), jnp.float32),
        ],
    )(s2p, cntp, state, wb2_t[:g2c], wb2_t[g2c:], bb2,
      fc1_t[:f], fc1_t[f:], b1, fc2_t, b2, mean_t, bm, ls_t, bl)

    return (mean, log_std)


# speedup vs baseline: 12.3274x; 1.0214x over previous
import functools

import jax
import jax.numpy as jnp
from jax import lax
from jax.experimental import pallas as pl
from jax.experimental.pallas import tpu as pltpu
from jax.experimental.pallas import tpu_sc as plsc

_NC = 2
_NS = 16
_LANES = 16

_SC_PARAMS = pltpu.CompilerParams(use_tc_tiling_on_sc=False,
                                  needs_layout_passes=False)


def _pick_chunk(epw: int) -> int:
    for c in range(128, 7, -8):
        if epw % c == 0 and (epw // c) % 2 == 0:
            return c
    raise ValueError(f"no valid chunk for {epw} edges per tile")


def _pick_sb(nch: int) -> int:
    for c in range(64, 1, -2):
        if nch % c == 0:
            return c
    return 2


def _pick_zr(rpt: int) -> int:
    for c in range(128, 0, -1):
        if rpt % c == 0:
            return c
    return 1


def _make_sc_edge_pass(n_nodes: int, d: int, n_edges: int, with_cnt: bool):
    assert n_edges % _NS == 0
    epw = n_edges // _NS
    chunk = _pick_chunk(epw)
    nch = epw // chunk
    sb = _pick_sb(nch)
    nsb = nch // sb
    assert n_nodes % _NS == 0
    rpt = n_nodes // _NS
    zr = _pick_zr(rpt)
    nz = rpt // zr

    mesh = plsc.VectorSubcoreMesh(
        core_axis_name="c", subcore_axis_name="s",
        num_cores=_NC, num_subcores=_NS)

    out_type = [jax.ShapeDtypeStruct((_NC, n_nodes, d), jnp.float32)]
    scratch = [
        pltpu.VMEM((sb, chunk), jnp.int32),
        pltpu.VMEM((sb, chunk), jnp.int32),
        pltpu.VMEM((chunk, d), jnp.float32),
        pltpu.VMEM((chunk, d), jnp.float32),
        pltpu.VMEM((chunk, d), jnp.float32),
        pltpu.VMEM((chunk, d), jnp.float32),
        pltpu.VMEM((chunk, d), jnp.float32),
        pltpu.VMEM((chunk, d), jnp.float32),
        pltpu.VMEM((zr, d), jnp.float32),
        pltpu.VMEM_SHARED((n_nodes, d), jnp.float32),
        pltpu.SemaphoreType.DMA,
        pltpu.SemaphoreType.DMA,
        pltpu.SemaphoreType.DMA,
        pltpu.SemaphoreType.DMA,
        pltpu.SemaphoreType.DMA,
        pltpu.SemaphoreType.DMA,
    ]
    if with_cnt:
        out_type.append(jax.ShapeDtypeStruct((_NS, n_nodes), jnp.float32))
        scratch.append(pltpu.VMEM((n_nodes,), jnp.float32))

    def body(*refs):
        tables = refs[:4]
        eidx_hbm, s_out = refs[4:6]
        rest = refs[6:]
        if with_cnt:
            c_out, rest = rest[0], rest[1:]
        (didx, sidx, a0, a1, b0, b1, c0, c1, zbuf, acc_sh,
         sga0, sga1, sgb0, sgb1, ss0, ss1) = rest[:16]
        cnt_v = rest[16] if with_cnt else None
        abuf = (a0, a1)
        bbuf = (b0, b1)
        cbuf = (c0, c1)
        sga = (sga0, sga1)
        sgb = (sgb0, sgb1)
        ss = (ss0, ss1)

        cid = lax.axis_index("c")
        sid = lax.axis_index("s")

        def zero_rows(r, _):
            for k in range(d // _LANES):
                zbuf[r, pl.ds(k * _LANES, _LANES)] = jnp.zeros((_LANES,), jnp.float32)
            return 0
        lax.fori_loop(0, zr, zero_rows, 0)
        for t in range(nz):
            pltpu.sync_copy(zbuf, acc_sh.at[pl.ds(sid * rpt + t * zr, zr)])
        if with_cnt:
            def zero_cnt(i, _):
                cnt_v[pl.ds(i * _LANES, _LANES)] = jnp.zeros((_LANES,), jnp.float32)
                return 0
            lax.fori_loop(0, n_nodes // _LANES, zero_cnt, 0)
        plsc.subcore_barrier()

        def issue_gathers(c, slot):
            di = didx.at[c]
            si = sidx.at[c]

            @pl.when(cid == 0)
            def _():
                pltpu.async_copy(tables[0].at[di], abuf[slot], sga[slot])
                pltpu.async_copy(tables[2].at[si], bbuf[slot], sgb[slot])

            @pl.when(cid != 0)
            def _():
                pltpu.async_copy(tables[1].at[di], abuf[slot], sga[slot])
                pltpu.async_copy(tables[3].at[si], bbuf[slot], sgb[slot])

        def wait_gathers(slot):
            pltpu.make_async_copy(tables[0].at[pl.ds(0, chunk)],
                                  abuf[slot], sga[slot]).wait()
            pltpu.make_async_copy(tables[2].at[pl.ds(0, chunk)],
                                  bbuf[slot], sgb[slot]).wait()

        def wait_scatter(slot):
            pltpu.make_async_copy(cbuf[slot], acc_sh.at[pl.ds(0, chunk)],
                                  ss[slot]).wait()

        def super_body(s, _):
            pltpu.sync_copy(eidx_hbm.at[1, sid, pl.ds(s * sb, sb)], didx)
            pltpu.sync_copy(eidx_hbm.at[0, sid, pl.ds(s * sb, sb)], sidx)
            issue_gathers(0, 0)

            def pair_body(t, _2):
                for b in (0, 1):
                    c = 2 * t + b
                    if b == 0:
                        issue_gathers(c + 1, 1)
                    else:
                        @pl.when(t < sb // 2 - 1)
                        def _():
                            issue_gathers(c + 1, 0)
                    wait_gathers(b)
                    @pl.when((s > 0) | (t > 0))
                    def _():
                        wait_scatter(b)
                    def row4(r4, _3):
                        base = r4 * 4
                        for rr in range(4):
                            r = base + rr
                            for k in range(d // _LANES):
                                sl = pl.ds(k * _LANES, _LANES)
                                cbuf[b][r, sl] = jnp.maximum(
                                    abuf[b][r, sl] + bbuf[b][r, sl], 0.0)
                        return 0
                    lax.fori_loop(0, chunk // 4, row4, 0)
                    pltpu.async_copy(cbuf[b], acc_sh.at[didx.at[c]], ss[b],
                                     add=True)
                    if with_cnt:
                        @pl.when(cid == 0)
                        def _():
                            for j in range(chunk // _LANES):
                                idx16 = didx[c, pl.ds(j * _LANES, _LANES)]
                                plsc.addupdate_scatter(
                                    cnt_v, [idx16],
                                    jnp.ones((_LANES,), jnp.float32))
                return 0
            lax.fori_loop(0, sb // 2, pair_body, 0)
            return 0
        lax.fori_loop(0, nsb, super_body, 0)

        wait_scatter(0)
        wait_scatter(1)
        plsc.subcore_barrier()

        pltpu.sync_copy(acc_sh.at[pl.ds(sid * rpt, rpt)],
                        s_out.at[cid, pl.ds(sid * rpt, rpt)])
        if with_cnt:
            @pl.when(cid == 0)
            def _():
                pltpu.sync_copy(cnt_v, c_out.at[sid])

    return pl.kernel(body, out_type=out_type, mesh=mesh, scratch_types=scratch,
                     compiler_params=_SC_PARAMS)


def _cnt_column(c_ref):
    ones = jnp.ones((_NS, 1), jnp.float32)
    return lax.dot_general(c_ref[...], ones, (((0,), (0,)), ((), ())),
                           preferred_element_type=jnp.float32)


def _proj_body(x_ref, wil_ref, wih_ref, wjl_ref, wjh_ref, bal_ref, bah_ref,
               pil_ref, pih_ref, pjl_ref, pjh_ref):
    xb = x_ref[...]
    pil_ref[...] = (jnp.dot(xb, wil_ref[...], preferred_element_type=jnp.float32)
                    + bal_ref[...])
    pih_ref[...] = (jnp.dot(xb, wih_ref[...], preferred_element_type=jnp.float32)
                    + bah_ref[...])
    pjl_ref[...] = jnp.dot(xb, wjl_ref[...], preferred_element_type=jnp.float32)
    pjh_ref[...] = jnp.dot(xb, wjh_ref[...], preferred_element_type=jnp.float32)


def _mid_body(s_ref, c_ref, wbl_ref, wbh_ref, bb_ref,
              wil_ref, wih_ref, wjl_ref, wjh_ref, bil_ref, bih_ref,
              qil_ref, qih_ref, qjl_ref, qjh_ref):
    cnt = _cnt_column(c_ref)
    inv = 1.0 / jnp.maximum(cnt, 1.0)
    ind = (cnt > 0.0).astype(jnp.float32)
    h = jnp.maximum(
        jnp.dot(s_ref[0] * inv, wbl_ref[...], preferred_element_type=jnp.float32)
        + jnp.dot(s_ref[1] * inv, wbh_ref[...], preferred_element_type=jnp.float32)
        + ind * bb_ref[...], 0.0)
    qil_ref[...] = (jnp.dot(h, wil_ref[...], preferred_element_type=jnp.float32)
                    + bil_ref[...])
    qih_ref[...] = (jnp.dot(h, wih_ref[...], preferred_element_type=jnp.float32)
                    + bih_ref[...])
    qjl_ref[...] = jnp.dot(h, wjl_ref[...], preferred_element_type=jnp.float32)
    qjh_ref[...] = jnp.dot(h, wjh_ref[...], preferred_element_type=jnp.float32)


def _final_body(n_nodes, s_ref, c_ref, st_ref, wb2l_ref, wb2h_ref, bb2_ref,
                f1s_ref, f1g_ref, b1_ref, f2_ref, b2_ref,
                wm_ref, bm_ref, wl_ref, bl_ref,
                mean_ref, ls_ref):
    cnt = _cnt_column(c_ref)
    inv = 1.0 / jnp.maximum(cnt, 1.0)
    ind = (cnt > 0.0).astype(jnp.float32)
    scale = 1.0 / n_nodes
    t_lo = jnp.sum(s_ref[0] * inv, axis=0, keepdims=True) * scale
    t_hi = jnp.sum(s_ref[1] * inv, axis=0, keepdims=True) * scale
    frac = jnp.sum(ind) * scale
    ge = (jnp.dot(t_lo, wb2l_ref[...], preferred_element_type=jnp.float32)
          + jnp.dot(t_hi, wb2h_ref[...], preferred_element_type=jnp.float32)
          + frac * bb2_ref[...])
    z = jnp.maximum(
        jnp.dot(st_ref[...], f1s_ref[...], preferred_element_type=jnp.float32)
        + jnp.dot(ge, f1g_ref[...], preferred_element_type=jnp.float32)
        + b1_ref[...], 0.0)
    z = jnp.maximum(
        jnp.dot(z, f2_ref[...], preferred_element_type=jnp.float32) + b2_ref[...], 0.0)
    mean_ref[...] = (jnp.dot(z, wm_ref[...], preferred_element_type=jnp.float32)
                     + bm_ref[...])
    ls_ref[...] = jnp.clip(
        jnp.dot(z, wl_ref[...], preferred_element_type=jnp.float32) + bl_ref[...],
        -20.0, 2.0)


def kernel(state, x, edge_index, g1_Wa, g1_ba, g1_Wb, g1_bb,
           g2_Wa, g2_ba, g2_Wb, g2_bb, fc1_W, fc1_b, fc2_W, fc2_b,
           mean_W, mean_b, ls_W, ls_b):
    n, f = x.shape
    hh = g1_Wa.shape[0]
    g = g2_Wa.shape[0]
    e = edge_index.shape[1]

    epw = e // _NS
    chunk = _pick_chunk(epw)
    nch = epw // chunk
    eidx = edge_index.astype(jnp.int32).reshape(2, _NS, nch, chunk)
    hh2 = hh // 2
    g2c = g // 2

    wi_t = g1_Wa[:, :f].T
    wj_t = g1_Wa[:, f:].T
    wb1_t = g1_Wb.T
    wa2i_t = g2_Wa[:, :hh].T
    wa2j_t = g2_Wa[:, hh:].T
    wb2_t = g2_Wb.T
    fc1_t, fc2_t = fc1_W.T, fc2_W.T
    mean_t, ls_t = mean_W.T, ls_W.T
    ba1 = g1_ba.reshape(1, hh)
    bb1 = g1_bb.reshape(1, hh)
    ba2 = g2_ba.reshape(1, g)
    bb2 = g2_bb.reshape(1, g)
    b1 = fc1_b.reshape(1, -1)
    b2 = fc2_b.reshape(1, -1)
    bm = mean_b.reshape(1, -1)
    bl = ls_b.reshape(1, -1)

    pshape = jax.ShapeDtypeStruct((n, hh2), jnp.float32)
    pi_lo, pi_hi, pj_lo, pj_hi = pl.pallas_call(
        _proj_body,
        out_shape=[pshape, pshape, pshape, pshape],
    )(x, wi_t[:, :hh2], wi_t[:, hh2:], wj_t[:, :hh2], wj_t[:, hh2:],
      ba1[:, :hh2], ba1[:, hh2:])

    s1p, cntp = _make_sc_edge_pass(n, hh2, e, with_cnt=True)(
        pi_lo, pi_hi, pj_lo, pj_hi, eidx)

    qshape = jax.ShapeDtypeStruct((n, g2c), jnp.float32)
    qi_lo, qi_hi, qj_lo, qj_hi = pl.pallas_call(
        _mid_body,
        out_shape=[qshape, qshape, qshape, qshape],
    )(s1p, cntp, wb1_t[:hh2], wb1_t[hh2:], bb1,
      wa2i_t[:, :g2c], wa2i_t[:, g2c:], wa2j_t[:, :g2c], wa2j_t[:, g2c:],
      ba2[:, :g2c], ba2[:, g2c:])

    (s2p,) = _make_sc_edge_pass(n, g2c, e, with_cnt=False)(
        qi_lo, qi_hi, qj_lo, qj_hi, eidx)

    mean, log_std = pl.pallas_call(
        functools.partial(_final_body, n),
        out_shape=[
            jax.ShapeDtypeStruct((1, mean_t.shape[1]), jnp.float32),
            jax.ShapeDtypeStruct((1, ls_t.shape[1]), jnp.float32),
        ],
    )(s2p, cntp, state, wb2_t[:g2c], wb2_t[g2c:], bb2,
      fc1_t[:f], fc1_t[f:], b1, fc2_t, b2, mean_t, bm, ls_t, bl)

    return (mean, log_std)


# speedup vs baseline: 12.8214x; 1.0401x over previous
import functools

import jax
import jax.numpy as jnp
from jax import lax
from jax.experimental import pallas as pl
from jax.experimental.pallas import tpu as pltpu
from jax.experimental.pallas import tpu_sc as plsc

_NC = 2
_NS = 16
_LANES = 16

_SC_PARAMS = pltpu.CompilerParams(use_tc_tiling_on_sc=False,
                                  needs_layout_passes=False)


def _pick_chunk(epw: int) -> int:
    for c in range(128, 7, -8):
        if epw % c == 0 and (epw // c) % 2 == 0:
            return c
    raise ValueError(f"no valid chunk for {epw} edges per tile")


def _pick_zr(rpt: int) -> int:
    for c in range(128, 0, -1):
        if rpt % c == 0:
            return c
    return 1


def _make_sc_edge_pass(n_nodes: int, d: int, n_edges: int, with_cnt: bool):
    assert n_edges % _NS == 0
    epw = n_edges // _NS
    chunk = _pick_chunk(epw)
    nch = epw // chunk
    assert n_nodes % _NS == 0
    rpt = n_nodes // _NS
    zr = _pick_zr(rpt)
    nz = rpt // zr

    mesh = plsc.VectorSubcoreMesh(
        core_axis_name="c", subcore_axis_name="s",
        num_cores=_NC, num_subcores=_NS)

    out_type = [jax.ShapeDtypeStruct((_NC, n_nodes, d), jnp.float32)]
    scratch = [
        pltpu.VMEM((nch, chunk), jnp.int32),
        pltpu.VMEM((nch, chunk), jnp.int32),
        pltpu.VMEM((chunk, d), jnp.float32),
        pltpu.VMEM((chunk, d), jnp.float32),
        pltpu.VMEM((chunk, d), jnp.float32),
        pltpu.VMEM((chunk, d), jnp.float32),
        pltpu.VMEM((chunk, d), jnp.float32),
        pltpu.VMEM((chunk, d), jnp.float32),
        pltpu.VMEM((zr, d), jnp.float32),
        pltpu.VMEM_SHARED((n_nodes, d), jnp.float32),
        pltpu.SemaphoreType.DMA,
        pltpu.SemaphoreType.DMA,
        pltpu.SemaphoreType.DMA,
        pltpu.SemaphoreType.DMA,
        pltpu.SemaphoreType.DMA,
        pltpu.SemaphoreType.DMA,
    ]
    if with_cnt:
        out_type.append(jax.ShapeDtypeStruct((_NS, n_nodes), jnp.float32))
        scratch.append(pltpu.VMEM((n_nodes,), jnp.float32))

    def body(*refs):
        tables = refs[:4]
        eidx_hbm, s_out = refs[4:6]
        rest = refs[6:]
        if with_cnt:
            c_out, rest = rest[0], rest[1:]
        (didx, sidx, a0, a1, b0, b1, c0, c1, zbuf, acc_sh,
         sga0, sga1, sgb0, sgb1, ss0, ss1) = rest[:16]
        cnt_v = rest[16] if with_cnt else None
        abuf = (a0, a1)
        bbuf = (b0, b1)
        cbuf = (c0, c1)
        sga = (sga0, sga1)
        sgb = (sgb0, sgb1)
        ss = (ss0, ss1)

        cid = lax.axis_index("c")
        sid = lax.axis_index("s")

        cp_di = pltpu.async_copy(eidx_hbm.at[1, sid], didx, sga0)
        cp_si = pltpu.async_copy(eidx_hbm.at[0, sid], sidx, sgb0)

        def zero_rows(r, _):
            for k in range(d // _LANES):
                zbuf[r, pl.ds(k * _LANES, _LANES)] = jnp.zeros((_LANES,), jnp.float32)
            return 0
        lax.fori_loop(0, zr, zero_rows, 0)
        for t in range(nz):
            pltpu.sync_copy(zbuf, acc_sh.at[pl.ds(sid * rpt + t * zr, zr)])
        if with_cnt:
            def zero_cnt(i, _):
                cnt_v[pl.ds(i * _LANES, _LANES)] = jnp.zeros((_LANES,), jnp.float32)
                return 0
            lax.fori_loop(0, n_nodes // _LANES, zero_cnt, 0)
        plsc.subcore_barrier()

        def issue_gathers(c, slot):
            di = didx.at[c]
            si = sidx.at[c]

            @pl.when(cid == 0)
            def _():
                pltpu.async_copy(tables[0].at[di], abuf[slot], sga[slot])
                pltpu.async_copy(tables[2].at[si], bbuf[slot], sgb[slot])

            @pl.when(cid != 0)
            def _():
                pltpu.async_copy(tables[1].at[di], abuf[slot], sga[slot])
                pltpu.async_copy(tables[3].at[si], bbuf[slot], sgb[slot])

        def wait_gathers(slot):
            pltpu.make_async_copy(tables[0].at[pl.ds(0, chunk)],
                                  abuf[slot], sga[slot]).wait()
            pltpu.make_async_copy(tables[2].at[pl.ds(0, chunk)],
                                  bbuf[slot], sgb[slot]).wait()

        def wait_scatter(slot):
            pltpu.make_async_copy(cbuf[slot], acc_sh.at[pl.ds(0, chunk)],
                                  ss[slot]).wait()

        cp_di.wait()
        cp_si.wait()
        issue_gathers(0, 0)

        def pair_body(t, _2):
            for b in (0, 1):
                c = 2 * t + b
                if b == 0:
                    issue_gathers(c + 1, 1)
                else:
                    @pl.when(t < nch // 2 - 1)
                    def _():
                        issue_gathers(c + 1, 0)
                wait_gathers(b)
                @pl.when(t > 0)
                def _():
                    wait_scatter(b)
                def row4(r4, _3):
                    base = r4 * 4
                    for rr in range(4):
                        r = base + rr
                        for k in range(d // _LANES):
                            sl = pl.ds(k * _LANES, _LANES)
                            cbuf[b][r, sl] = jnp.maximum(
                                abuf[b][r, sl] + bbuf[b][r, sl], 0.0)
                    return 0
                lax.fori_loop(0, chunk // 4, row4, 0)
                pltpu.async_copy(cbuf[b], acc_sh.at[didx.at[c]], ss[b],
                                 add=True)
                if with_cnt:
                    @pl.when(cid == 0)
                    def _():
                        for j in range(chunk // _LANES):
                            idx16 = didx[c, pl.ds(j * _LANES, _LANES)]
                            plsc.addupdate_scatter(
                                cnt_v, [idx16],
                                jnp.ones((_LANES,), jnp.float32))
            return 0
        lax.fori_loop(0, nch // 2, pair_body, 0)

        wait_scatter(0)
        wait_scatter(1)
        plsc.subcore_barrier()

        pltpu.sync_copy(acc_sh.at[pl.ds(sid * rpt, rpt)],
                        s_out.at[cid, pl.ds(sid * rpt, rpt)])
        if with_cnt:
            @pl.when(cid == 0)
            def _():
                pltpu.sync_copy(cnt_v, c_out.at[sid])

    return pl.kernel(body, out_type=out_type, mesh=mesh, scratch_types=scratch,
                     compiler_params=_SC_PARAMS)


def _cnt_column(c_ref):
    ones = jnp.ones((_NS, 1), jnp.float32)
    return lax.dot_general(c_ref[...], ones, (((0,), (0,)), ((), ())),
                           preferred_element_type=jnp.float32)


def _proj_body(x_ref, wil_ref, wih_ref, wjl_ref, wjh_ref, bal_ref, bah_ref,
               pil_ref, pih_ref, pjl_ref, pjh_ref):
    xb = x_ref[...]
    pil_ref[...] = (jnp.dot(xb, wil_ref[...], preferred_element_type=jnp.float32)
                    + bal_ref[...])
    pih_ref[...] = (jnp.dot(xb, wih_ref[...], preferred_element_type=jnp.float32)
                    + bah_ref[...])
    pjl_ref[...] = jnp.dot(xb, wjl_ref[...], preferred_element_type=jnp.float32)
    pjh_ref[...] = jnp.dot(xb, wjh_ref[...], preferred_element_type=jnp.float32)


def _mid_body(s_ref, c_ref, wbl_ref, wbh_ref, bb_ref,
              wil_ref, wih_ref, wjl_ref, wjh_ref, bil_ref, bih_ref,
              qil_ref, qih_ref, qjl_ref, qjh_ref):
    cnt = _cnt_column(c_ref)
    inv = 1.0 / jnp.maximum(cnt, 1.0)
    ind = (cnt > 0.0).astype(jnp.float32)
    h = jnp.maximum(
        jnp.dot(s_ref[0] * inv, wbl_ref[...], preferred_element_type=jnp.float32)
        + jnp.dot(s_ref[1] * inv, wbh_ref[...], preferred_element_type=jnp.float32)
        + ind * bb_ref[...], 0.0)
    qil_ref[...] = (jnp.dot(h, wil_ref[...], preferred_element_type=jnp.float32)
                    + bil_ref[...])
    qih_ref[...] = (jnp.dot(h, wih_ref[...], preferred_element_type=jnp.float32)
                    + bih_ref[...])
    qjl_ref[...] = jnp.dot(h, wjl_ref[...], preferred_element_type=jnp.float32)
    qjh_ref[...] = jnp.dot(h, wjh_ref[...], preferred_element_type=jnp.float32)


def _final_body(n_nodes, s_ref, c_ref, st_ref, wb2l_ref, wb2h_ref, bb2_ref,
                f1s_ref, f1g_ref, b1_ref, f2_ref, b2_ref,
                wm_ref, bm_ref, wl_ref, bl_ref,
                mean_ref, ls_ref):
    cnt = _cnt_column(c_ref)
    inv = 1.0 / jnp.maximum(cnt, 1.0)
    ind = (cnt > 0.0).astype(jnp.float32)
    scale = 1.0 / n_nodes
    t_lo = jnp.sum(s_ref[0] * inv, axis=0, keepdims=True) * scale
    t_hi = jnp.sum(s_ref[1] * inv, axis=0, keepdims=True) * scale
    frac = jnp.sum(ind) * scale
    ge = (jnp.dot(t_lo, wb2l_ref[...], preferred_element_type=jnp.float32)
          + jnp.dot(t_hi, wb2h_ref[...], preferred_element_type=jnp.float32)
          + frac * bb2_ref[...])
    z = jnp.maximum(
        jnp.dot(st_ref[...], f1s_ref[...], preferred_element_type=jnp.float32)
        + jnp.dot(ge, f1g_ref[...], preferred_element_type=jnp.float32)
        + b1_ref[...], 0.0)
    z = jnp.maximum(
        jnp.dot(z, f2_ref[...], preferred_element_type=jnp.float32) + b2_ref[...], 0.0)
    mean_ref[...] = (jnp.dot(z, wm_ref[...], preferred_element_type=jnp.float32)
                     + bm_ref[...])
    ls_ref[...] = jnp.clip(
        jnp.dot(z, wl_ref[...], preferred_element_type=jnp.float32) + bl_ref[...],
        -20.0, 2.0)


def kernel(state, x, edge_index, g1_Wa, g1_ba, g1_Wb, g1_bb,
           g2_Wa, g2_ba, g2_Wb, g2_bb, fc1_W, fc1_b, fc2_W, fc2_b,
           mean_W, mean_b, ls_W, ls_b):
    n, f = x.shape
    hh = g1_Wa.shape[0]
    g = g2_Wa.shape[0]
    e = edge_index.shape[1]

    epw = e // _NS
    chunk = _pick_chunk(epw)
    nch = epw // chunk
    eidx = edge_index.astype(jnp.int32).reshape(2, _NS, nch, chunk)
    hh2 = hh // 2
    g2c = g // 2

    wi_t = g1_Wa[:, :f].T
    wj_t = g1_Wa[:, f:].T
    wb1_t = g1_Wb.T
    wa2i_t = g2_Wa[:, :hh].T
    wa2j_t = g2_Wa[:, hh:].T
    wb2_t = g2_Wb.T
    fc1_t, fc2_t = fc1_W.T, fc2_W.T
    mean_t, ls_t = mean_W.T, ls_W.T
    ba1 = g1_ba.reshape(1, hh)
    bb1 = g1_bb.reshape(1, hh)
    ba2 = g2_ba.reshape(1, g)
    bb2 = g2_bb.reshape(1, g)
    b1 = fc1_b.reshape(1, -1)
    b2 = fc2_b.reshape(1, -1)
    bm = mean_b.reshape(1, -1)
    bl = ls_b.reshape(1, -1)

    pshape = jax.ShapeDtypeStruct((n, hh2), jnp.float32)
    pi_lo, pi_hi, pj_lo, pj_hi = pl.pallas_call(
        _proj_body,
        out_shape=[pshape, pshape, pshape, pshape],
    )(x, wi_t[:, :hh2], wi_t[:, hh2:], wj_t[:, :hh2], wj_t[:, hh2:],
      ba1[:, :hh2], ba1[:, hh2:])

    s1p, cntp = _make_sc_edge_pass(n, hh2, e, with_cnt=True)(
        pi_lo, pi_hi, pj_lo, pj_hi, eidx)

    qshape = jax.ShapeDtypeStruct((n, g2c), jnp.float32)
    qi_lo, qi_hi, qj_lo, qj_hi = pl.pallas_call(
        _mid_body,
        out_shape=[qshape, qshape, qshape, qshape],
    )(s1p, cntp, wb1_t[:hh2], wb1_t[hh2:], bb1,
      wa2i_t[:, :g2c], wa2i_t[:, g2c:], wa2j_t[:, :g2c], wa2j_t[:, g2c:],
      ba2[:, :g2c], ba2[:, g2c:])

    (s2p,) = _make_sc_edge_pass(n, g2c, e, with_cnt=False)(
        qi_lo, qi_hi, qj_lo, qj_hi, eidx)

    mean, log_std = pl.pallas_call(
        functools.partial(_final_body, n),
        out_shape=[
            jax.ShapeDtypeStruct((1, mean_t.shape[1]), jnp.float32),
            jax.ShapeDtypeStruct((1, ls_t.shape[1]), jnp.float32),
        ],
    )(s2p, cntp, state, wb2_t[:g2c], wb2_t[g2c:], bb2,
      fc1_t[:f], fc1_t[f:], b1, fc2_t, b2, mean_t, bm, ls_t, bl)

    return (mean, log_std)


# speedup vs baseline: 15.5302x; 1.2113x over previous
import functools

import jax
import jax.numpy as jnp
from jax import lax
from jax.experimental import pallas as pl
from jax.experimental.pallas import tpu as pltpu
from jax.experimental.pallas import tpu_sc as plsc

_NC = 2
_NS = 16
_LANES = 16

_SC_PARAMS = pltpu.CompilerParams(use_tc_tiling_on_sc=False,
                                  needs_layout_passes=False)


def _pick_chunk(epw: int) -> int:
    for c in range(128, 7, -8):
        if epw % c == 0 and (epw // c) % 2 == 0:
            return c
    raise ValueError(f"no valid chunk for {epw} edges per tile")


def _pick_zr(rpt: int) -> int:
    for c in range(128, 0, -1):
        if rpt % c == 0:
            return c
    return 1


def _make_sc_edge_pass(n_nodes: int, d: int, n_edges: int, with_cnt: bool):
    assert n_edges % _NS == 0
    epw = n_edges // _NS
    chunk = _pick_chunk(epw)
    nch = epw // chunk
    assert n_nodes % _NS == 0
    rpt = n_nodes // _NS
    zr = _pick_zr(rpt)
    nz = rpt // zr

    mesh = plsc.VectorSubcoreMesh(
        core_axis_name="c", subcore_axis_name="s",
        num_cores=_NC, num_subcores=_NS)

    out_type = [jax.ShapeDtypeStruct((_NC, n_nodes, d), jnp.bfloat16)]
    scratch = [
        pltpu.VMEM((nch, chunk), jnp.int32),
        pltpu.VMEM((nch, chunk), jnp.int32),
        pltpu.VMEM((chunk, d), jnp.bfloat16),
        pltpu.VMEM((chunk, d), jnp.bfloat16),
        pltpu.VMEM((chunk, d), jnp.bfloat16),
        pltpu.VMEM((chunk, d), jnp.bfloat16),
        pltpu.VMEM((chunk, d), jnp.bfloat16),
        pltpu.VMEM((chunk, d), jnp.bfloat16),
        pltpu.VMEM((zr, d), jnp.bfloat16),
        pltpu.VMEM_SHARED((n_nodes, d), jnp.bfloat16),
        pltpu.SemaphoreType.DMA,
        pltpu.SemaphoreType.DMA,
        pltpu.SemaphoreType.DMA,
        pltpu.SemaphoreType.DMA,
        pltpu.SemaphoreType.DMA,
        pltpu.SemaphoreType.DMA,
    ]
    if with_cnt:
        out_type.append(jax.ShapeDtypeStruct((_NS, n_nodes), jnp.float32))
        scratch.append(pltpu.VMEM((n_nodes,), jnp.float32))

    def body(*refs):
        tables = refs[:4]
        eidx_hbm, s_out = refs[4:6]
        rest = refs[6:]
        if with_cnt:
            c_out, rest = rest[0], rest[1:]
        (didx, sidx, a0, a1, b0, b1, c0, c1, zbuf, acc_sh,
         sga0, sga1, sgb0, sgb1, ss0, ss1) = rest[:16]
        cnt_v = rest[16] if with_cnt else None
        abuf = (a0, a1)
        bbuf = (b0, b1)
        cbuf = (c0, c1)
        sga = (sga0, sga1)
        sgb = (sgb0, sgb1)
        ss = (ss0, ss1)

        cid = lax.axis_index("c")
        sid = lax.axis_index("s")

        cp_di = pltpu.async_copy(eidx_hbm.at[1, sid], didx, sga0)
        cp_si = pltpu.async_copy(eidx_hbm.at[0, sid], sidx, sgb0)

        def zero_rows(r, _):
            for k in range(d // (2 * _LANES)):
                sl = pl.ds(k * 2 * _LANES, 2 * _LANES)
                zbuf[r, sl] = jnp.zeros((2 * _LANES,), jnp.bfloat16)
            return 0
        lax.fori_loop(0, zr, zero_rows, 0)
        for t in range(nz):
            pltpu.sync_copy(zbuf, acc_sh.at[pl.ds(sid * rpt + t * zr, zr)])
        if with_cnt:
            def zero_cnt(i, _):
                cnt_v[pl.ds(i * _LANES, _LANES)] = jnp.zeros((_LANES,), jnp.float32)
                return 0
            lax.fori_loop(0, n_nodes // _LANES, zero_cnt, 0)
        plsc.subcore_barrier()

        def issue_gathers(c, slot):
            di = didx.at[c]
            si = sidx.at[c]

            @pl.when(cid == 0)
            def _():
                pltpu.async_copy(tables[0].at[di], abuf[slot], sga[slot])
                pltpu.async_copy(tables[2].at[si], bbuf[slot], sgb[slot])

            @pl.when(cid != 0)
            def _():
                pltpu.async_copy(tables[1].at[di], abuf[slot], sga[slot])
                pltpu.async_copy(tables[3].at[si], bbuf[slot], sgb[slot])

        def wait_gathers(slot):
            pltpu.make_async_copy(tables[0].at[pl.ds(0, chunk)],
                                  abuf[slot], sga[slot]).wait()
            pltpu.make_async_copy(tables[2].at[pl.ds(0, chunk)],
                                  bbuf[slot], sgb[slot]).wait()

        def wait_scatter(slot):
            pltpu.make_async_copy(cbuf[slot], acc_sh.at[pl.ds(0, chunk)],
                                  ss[slot]).wait()

        cp_di.wait()
        cp_si.wait()
        issue_gathers(0, 0)

        def pair_body(t, _2):
            for b in (0, 1):
                c = 2 * t + b
                if b == 0:
                    issue_gathers(c + 1, 1)
                else:
                    @pl.when(t < nch // 2 - 1)
                    def _():
                        issue_gathers(c + 1, 0)
                wait_gathers(b)
                @pl.when(t > 0)
                def _():
                    wait_scatter(b)
                def row4(r4, _3):
                    base = r4 * 4
                    for rr in range(4):
                        r = base + rr
                        for k in range(d // (2 * _LANES)):
                            sl = pl.ds(k * 2 * _LANES, 2 * _LANES)
                            cbuf[b][r, sl] = jnp.maximum(
                                abuf[b][r, sl] + bbuf[b][r, sl], 0.0)
                    return 0
                lax.fori_loop(0, chunk // 4, row4, 0)
                pltpu.async_copy(cbuf[b], acc_sh.at[didx.at[c]], ss[b],
                                 add=True)
                if with_cnt:
                    @pl.when(cid == 0)
                    def _():
                        for j in range(chunk // _LANES):
                            idx16 = didx[c, pl.ds(j * _LANES, _LANES)]
                            plsc.addupdate_scatter(
                                cnt_v, [idx16],
                                jnp.ones((_LANES,), jnp.float32))
            return 0
        lax.fori_loop(0, nch // 2, pair_body, 0)

        wait_scatter(0)
        wait_scatter(1)
        plsc.subcore_barrier()

        pltpu.sync_copy(acc_sh.at[pl.ds(sid * rpt, rpt)],
                        s_out.at[cid, pl.ds(sid * rpt, rpt)])
        if with_cnt:
            @pl.when(cid == 0)
            def _():
                pltpu.sync_copy(cnt_v, c_out.at[sid])

    return pl.kernel(body, out_type=out_type, mesh=mesh, scratch_types=scratch,
                     compiler_params=_SC_PARAMS)


def _cnt_column(c_ref):
    ones = jnp.ones((_NS, 1), jnp.float32)
    return lax.dot_general(c_ref[...], ones, (((0,), (0,)), ((), ())),
                           preferred_element_type=jnp.float32)


def _proj_body(x_ref, wil_ref, wih_ref, wjl_ref, wjh_ref, bal_ref, bah_ref,
               pil_ref, pih_ref, pjl_ref, pjh_ref):
    xb = x_ref[...]
    pil_ref[...] = (jnp.dot(xb, wil_ref[...], preferred_element_type=jnp.float32)
                    + bal_ref[...]).astype(jnp.bfloat16)
    pih_ref[...] = (jnp.dot(xb, wih_ref[...], preferred_element_type=jnp.float32)
                    + bah_ref[...]).astype(jnp.bfloat16)
    pjl_ref[...] = jnp.dot(
        xb, wjl_ref[...], preferred_element_type=jnp.float32).astype(jnp.bfloat16)
    pjh_ref[...] = jnp.dot(
        xb, wjh_ref[...], preferred_element_type=jnp.float32).astype(jnp.bfloat16)


def _mid_body(s_ref, c_ref, wbl_ref, wbh_ref, bb_ref,
              wil_ref, wih_ref, wjl_ref, wjh_ref, bil_ref, bih_ref,
              qil_ref, qih_ref, qjl_ref, qjh_ref):
    cnt = _cnt_column(c_ref)
    inv = 1.0 / jnp.maximum(cnt, 1.0)
    ind = (cnt > 0.0).astype(jnp.float32)
    s_lo = s_ref[0].astype(jnp.float32)
    s_hi = s_ref[1].astype(jnp.float32)
    h = jnp.maximum(
        jnp.dot(s_lo * inv, wbl_ref[...], preferred_element_type=jnp.float32)
        + jnp.dot(s_hi * inv, wbh_ref[...], preferred_element_type=jnp.float32)
        + ind * bb_ref[...], 0.0)
    qil_ref[...] = (jnp.dot(h, wil_ref[...], preferred_element_type=jnp.float32)
                    + bil_ref[...]).astype(jnp.bfloat16)
    qih_ref[...] = (jnp.dot(h, wih_ref[...], preferred_element_type=jnp.float32)
                    + bih_ref[...]).astype(jnp.bfloat16)
    qjl_ref[...] = jnp.dot(
        h, wjl_ref[...], preferred_element_type=jnp.float32).astype(jnp.bfloat16)
    qjh_ref[...] = jnp.dot(
        h, wjh_ref[...], preferred_element_type=jnp.float32).astype(jnp.bfloat16)


def _final_body(n_nodes, s_ref, c_ref, st_ref, wb2l_ref, wb2h_ref, bb2_ref,
                f1s_ref, f1g_ref, b1_ref, f2_ref, b2_ref,
                wm_ref, bm_ref, wl_ref, bl_ref,
                mean_ref, ls_ref):
    cnt = _cnt_column(c_ref)
    inv = 1.0 / jnp.maximum(cnt, 1.0)
    ind = (cnt > 0.0).astype(jnp.float32)
    scale = 1.0 / n_nodes
    t_lo = jnp.sum(s_ref[0].astype(jnp.float32) * inv, axis=0, keepdims=True) * scale
    t_hi = jnp.sum(s_ref[1].astype(jnp.float32) * inv, axis=0, keepdims=True) * scale
    frac = jnp.sum(ind) * scale
    ge = (jnp.dot(t_lo, wb2l_ref[...], preferred_element_type=jnp.float32)
          + jnp.dot(t_hi, wb2h_ref[...], preferred_element_type=jnp.float32)
          + frac * bb2_ref[...])
    z = jnp.maximum(
        jnp.dot(st_ref[...], f1s_ref[...], preferred_element_type=jnp.float32)
        + jnp.dot(ge, f1g_ref[...], preferred_element_type=jnp.float32)
        + b1_ref[...], 0.0)
    z = jnp.maximum(
        jnp.dot(z, f2_ref[...], preferred_element_type=jnp.float32) + b2_ref[...], 0.0)
    mean_ref[...] = (jnp.dot(z, wm_ref[...], preferred_element_type=jnp.float32)
                     + bm_ref[...])
    ls_ref[...] = jnp.clip(
        jnp.dot(z, wl_ref[...], preferred_element_type=jnp.float32) + bl_ref[...],
        -20.0, 2.0)


def kernel(state, x, edge_index, g1_Wa, g1_ba, g1_Wb, g1_bb,
           g2_Wa, g2_ba, g2_Wb, g2_bb, fc1_W, fc1_b, fc2_W, fc2_b,
           mean_W, mean_b, ls_W, ls_b):
    n, f = x.shape
    hh = g1_Wa.shape[0]
    g = g2_Wa.shape[0]
    e = edge_index.shape[1]

    epw = e // _NS
    chunk = _pick_chunk(epw)
    nch = epw // chunk
    eidx = edge_index.astype(jnp.int32).reshape(2, _NS, nch, chunk)
    hh2 = hh // 2
    g2c = g // 2

    wi_t = g1_Wa[:, :f].T
    wj_t = g1_Wa[:, f:].T
    wb1_t = g1_Wb.T
    wa2i_t = g2_Wa[:, :hh].T
    wa2j_t = g2_Wa[:, hh:].T
    wb2_t = g2_Wb.T
    fc1_t, fc2_t = fc1_W.T, fc2_W.T
    mean_t, ls_t = mean_W.T, ls_W.T
    ba1 = g1_ba.reshape(1, hh)
    bb1 = g1_bb.reshape(1, hh)
    ba2 = g2_ba.reshape(1, g)
    bb2 = g2_bb.reshape(1, g)
    b1 = fc1_b.reshape(1, -1)
    b2 = fc2_b.reshape(1, -1)
    bm = mean_b.reshape(1, -1)
    bl = ls_b.reshape(1, -1)

    pshape = jax.ShapeDtypeStruct((n, hh2), jnp.bfloat16)
    pi_lo, pi_hi, pj_lo, pj_hi = pl.pallas_call(
        _proj_body,
        out_shape=[pshape, pshape, pshape, pshape],
    )(x, wi_t[:, :hh2], wi_t[:, hh2:], wj_t[:, :hh2], wj_t[:, hh2:],
      ba1[:, :hh2], ba1[:, hh2:])

    s1p, cntp = _make_sc_edge_pass(n, hh2, e, with_cnt=True)(
        pi_lo, pi_hi, pj_lo, pj_hi, eidx)

    qshape = jax.ShapeDtypeStruct((n, g2c), jnp.bfloat16)
    qi_lo, qi_hi, qj_lo, qj_hi = pl.pallas_call(
        _mid_body,
        out_shape=[qshape, qshape, qshape, qshape],
    )(s1p, cntp, wb1_t[:hh2], wb1_t[hh2:], bb1,
      wa2i_t[:, :g2c], wa2i_t[:, g2c:], wa2j_t[:, :g2c], wa2j_t[:, g2c:],
      ba2[:, :g2c], ba2[:, g2c:])

    (s2p,) = _make_sc_edge_pass(n, g2c, e, with_cnt=False)(
        qi_lo, qi_hi, qj_lo, qj_hi, eidx)

    mean, log_std = pl.pallas_call(
        functools.partial(_final_body, n),
        out_shape=[
            jax.ShapeDtypeStruct((1, mean_t.shape[1]), jnp.float32),
            jax.ShapeDtypeStruct((1, ls_t.shape[1]), jnp.float32),
        ],
    )(s2p, cntp, state, wb2_t[:g2c], wb2_t[g2c:], bb2,
      fc1_t[:f], fc1_t[f:], b1, fc2_t, b2, mean_t, bm, ls_t, bl)

    return (mean, log_std)


# speedup vs baseline: 15.7238x; 1.0125x over previous
import functools

import jax
import jax.numpy as jnp
from jax import lax
from jax.experimental import pallas as pl
from jax.experimental.pallas import tpu as pltpu
from jax.experimental.pallas import tpu_sc as plsc

_NC = 2
_NS = 16
_LANES = 16

_SC_PARAMS = pltpu.CompilerParams(use_tc_tiling_on_sc=False,
                                  needs_layout_passes=False)


_CHUNK = 128


def _edge_geom(epw: int):
    nch = -(-epw // _CHUNK)
    if nch % 2:
        nch += 1
    full = epw // _CHUNK
    rem = epw - full * _CHUNK
    assert rem % _LANES == 0
    return nch, full, rem


def _pick_zr(rpt: int) -> int:
    for c in range(128, 0, -1):
        if rpt % c == 0:
            return c
    return 1


def _make_sc_edge_pass(n_nodes: int, d: int, n_edges: int, with_cnt: bool):
    assert n_edges % _NS == 0
    epw = n_edges // _NS
    chunk = _CHUNK
    nch, full, rem = _edge_geom(epw)
    assert n_nodes % _NS == 0
    rpt = n_nodes // _NS
    zr = _pick_zr(rpt)
    nz = rpt // zr

    mesh = plsc.VectorSubcoreMesh(
        core_axis_name="c", subcore_axis_name="s",
        num_cores=_NC, num_subcores=_NS)

    out_type = [jax.ShapeDtypeStruct((_NC, n_nodes, d), jnp.bfloat16)]
    scratch = [
        pltpu.VMEM((nch, chunk), jnp.int32),
        pltpu.VMEM((nch, chunk), jnp.int32),
        pltpu.VMEM((chunk, d), jnp.bfloat16),
        pltpu.VMEM((chunk, d), jnp.bfloat16),
        pltpu.VMEM((chunk, d), jnp.bfloat16),
        pltpu.VMEM((chunk, d), jnp.bfloat16),
        pltpu.VMEM((chunk, d), jnp.bfloat16),
        pltpu.VMEM((chunk, d), jnp.bfloat16),
        pltpu.VMEM((zr, d), jnp.bfloat16),
        pltpu.VMEM_SHARED((n_nodes, d), jnp.bfloat16),
        pltpu.SemaphoreType.DMA,
        pltpu.SemaphoreType.DMA,
        pltpu.SemaphoreType.DMA,
        pltpu.SemaphoreType.DMA,
        pltpu.SemaphoreType.DMA,
        pltpu.SemaphoreType.DMA,
    ]
    if with_cnt:
        out_type.append(jax.ShapeDtypeStruct((_NS, n_nodes), jnp.float32))
        scratch.append(pltpu.VMEM((n_nodes,), jnp.float32))

    def body(*refs):
        tables = refs[:4]
        eidx_hbm, s_out = refs[4:6]
        rest = refs[6:]
        if with_cnt:
            c_out, rest = rest[0], rest[1:]
        (didx, sidx, a0, a1, b0, b1, c0, c1, zbuf, acc_sh,
         sga0, sga1, sgb0, sgb1, ss0, ss1) = rest[:16]
        cnt_v = rest[16] if with_cnt else None
        abuf = (a0, a1)
        bbuf = (b0, b1)
        cbuf = (c0, c1)
        sga = (sga0, sga1)
        sgb = (sgb0, sgb1)
        ss = (ss0, ss1)

        cid = lax.axis_index("c")
        sid = lax.axis_index("s")

        cp_di = pltpu.async_copy(eidx_hbm.at[1, sid], didx, sga0)
        cp_si = pltpu.async_copy(eidx_hbm.at[0, sid], sidx, sgb0)

        def zero_rows(r, _):
            for k in range(d // (2 * _LANES)):
                sl = pl.ds(k * 2 * _LANES, 2 * _LANES)
                zbuf[r, sl] = jnp.zeros((2 * _LANES,), jnp.bfloat16)
            return 0
        lax.fori_loop(0, zr, zero_rows, 0)
        for t in range(nz):
            pltpu.sync_copy(zbuf, acc_sh.at[pl.ds(sid * rpt + t * zr, zr)])
        if with_cnt:
            def zero_cnt(i, _):
                cnt_v[pl.ds(i * _LANES, _LANES)] = jnp.zeros((_LANES,), jnp.float32)
                return 0
            lax.fori_loop(0, n_nodes // _LANES, zero_cnt, 0)
        plsc.subcore_barrier()

        def issue_gathers(c, slot):
            di = didx.at[c]
            si = sidx.at[c]

            @pl.when(cid == 0)
            def _():
                pltpu.async_copy(tables[0].at[di], abuf[slot], sga[slot])
                pltpu.async_copy(tables[2].at[si], bbuf[slot], sgb[slot])

            @pl.when(cid != 0)
            def _():
                pltpu.async_copy(tables[1].at[di], abuf[slot], sga[slot])
                pltpu.async_copy(tables[3].at[si], bbuf[slot], sgb[slot])

        def wait_gathers(slot):
            pltpu.make_async_copy(tables[0].at[pl.ds(0, chunk)],
                                  abuf[slot], sga[slot]).wait()
            pltpu.make_async_copy(tables[2].at[pl.ds(0, chunk)],
                                  bbuf[slot], sgb[slot]).wait()

        def wait_scatter(slot):
            pltpu.make_async_copy(cbuf[slot], acc_sh.at[pl.ds(0, chunk)],
                                  ss[slot]).wait()

        cp_di.wait()
        cp_si.wait()
        issue_gathers(0, 0)

        def pair_body(t, _2):
            for b in (0, 1):
                c = 2 * t + b
                if b == 0:
                    issue_gathers(c + 1, 1)
                else:
                    @pl.when(t < nch // 2 - 1)
                    def _():
                        issue_gathers(c + 1, 0)
                wait_gathers(b)
                @pl.when(t > 0)
                def _():
                    wait_scatter(b)
                def row4(r4, _3):
                    base = r4 * 4
                    for rr in range(4):
                        r = base + rr
                        for k in range(d // (2 * _LANES)):
                            sl = pl.ds(k * 2 * _LANES, 2 * _LANES)
                            cbuf[b][r, sl] = jnp.maximum(
                                abuf[b][r, sl] + bbuf[b][r, sl], 0.0)
                    return 0
                lax.fori_loop(0, chunk // 4, row4, 0)

                def zero_crow(r, _3):
                    for k in range(d // (2 * _LANES)):
                        sl = pl.ds(k * 2 * _LANES, 2 * _LANES)
                        cbuf[b][r, sl] = jnp.zeros((2 * _LANES,), jnp.bfloat16)
                    return 0
                if rem:
                    @pl.when(c == full)
                    def _():
                        lax.fori_loop(rem, chunk, zero_crow, 0)

                @pl.when(c > full if rem else c >= full)
                def _():
                    lax.fori_loop(0, chunk, zero_crow, 0)

                pltpu.async_copy(cbuf[b], acc_sh.at[didx.at[c]], ss[b],
                                 add=True)
                if with_cnt:
                    @pl.when((cid == 0) & (c < full))
                    def _():
                        for j in range(chunk // _LANES):
                            idx16 = didx[c, pl.ds(j * _LANES, _LANES)]
                            plsc.addupdate_scatter(
                                cnt_v, [idx16],
                                jnp.ones((_LANES,), jnp.float32))
                    if rem:
                        @pl.when((cid == 0) & (c == full))
                        def _():
                            for j in range(rem // _LANES):
                                idx16 = didx[c, pl.ds(j * _LANES, _LANES)]
                                plsc.addupdate_scatter(
                                    cnt_v, [idx16],
                                    jnp.ones((_LANES,), jnp.float32))
            return 0
        lax.fori_loop(0, nch // 2, pair_body, 0)

        wait_scatter(0)
        wait_scatter(1)
        plsc.subcore_barrier()

        pltpu.sync_copy(acc_sh.at[pl.ds(sid * rpt, rpt)],
                        s_out.at[cid, pl.ds(sid * rpt, rpt)])
        if with_cnt:
            @pl.when(cid == 0)
            def _():
                pltpu.sync_copy(cnt_v, c_out.at[sid])

    return pl.kernel(body, out_type=out_type, mesh=mesh, scratch_types=scratch,
                     compiler_params=_SC_PARAMS)


def _cnt_column(c_ref):
    ones = jnp.ones((_NS, 1), jnp.float32)
    return lax.dot_general(c_ref[...], ones, (((0,), (0,)), ((), ())),
                           preferred_element_type=jnp.float32)


def _proj_body(x_ref, wil_ref, wih_ref, wjl_ref, wjh_ref, bal_ref, bah_ref,
               pil_ref, pih_ref, pjl_ref, pjh_ref):
    xb = x_ref[...]
    pil_ref[...] = (jnp.dot(xb, wil_ref[...], preferred_element_type=jnp.float32)
                    + bal_ref[...]).astype(jnp.bfloat16)
    pih_ref[...] = (jnp.dot(xb, wih_ref[...], preferred_element_type=jnp.float32)
                    + bah_ref[...]).astype(jnp.bfloat16)
    pjl_ref[...] = jnp.dot(
        xb, wjl_ref[...], preferred_element_type=jnp.float32).astype(jnp.bfloat16)
    pjh_ref[...] = jnp.dot(
        xb, wjh_ref[...], preferred_element_type=jnp.float32).astype(jnp.bfloat16)


def _mid_body(s_ref, c_ref, wbl_ref, wbh_ref, bb_ref,
              wil_ref, wih_ref, wjl_ref, wjh_ref, bil_ref, bih_ref,
              qil_ref, qih_ref, qjl_ref, qjh_ref):
    cnt = _cnt_column(c_ref)
    inv = 1.0 / jnp.maximum(cnt, 1.0)
    ind = (cnt > 0.0).astype(jnp.float32)
    s_lo = s_ref[0].astype(jnp.float32)
    s_hi = s_ref[1].astype(jnp.float32)
    h = jnp.maximum(
        jnp.dot(s_lo * inv, wbl_ref[...], preferred_element_type=jnp.float32)
        + jnp.dot(s_hi * inv, wbh_ref[...], preferred_element_type=jnp.float32)
        + ind * bb_ref[...], 0.0)
    qil_ref[...] = (jnp.dot(h, wil_ref[...], preferred_element_type=jnp.float32)
                    + bil_ref[...]).astype(jnp.bfloat16)
    qih_ref[...] = (jnp.dot(h, wih_ref[...], preferred_element_type=jnp.float32)
                    + bih_ref[...]).astype(jnp.bfloat16)
    qjl_ref[...] = jnp.dot(
        h, wjl_ref[...], preferred_element_type=jnp.float32).astype(jnp.bfloat16)
    qjh_ref[...] = jnp.dot(
        h, wjh_ref[...], preferred_element_type=jnp.float32).astype(jnp.bfloat16)


def _final_body(n_nodes, s_ref, c_ref, st_ref, wb2l_ref, wb2h_ref, bb2_ref,
                f1s_ref, f1g_ref, b1_ref, f2_ref, b2_ref,
                wm_ref, bm_ref, wl_ref, bl_ref,
                mean_ref, ls_ref):
    cnt = _cnt_column(c_ref)
    inv = 1.0 / jnp.maximum(cnt, 1.0)
    ind = (cnt > 0.0).astype(jnp.float32)
    scale = 1.0 / n_nodes
    t_lo = jnp.sum(s_ref[0].astype(jnp.float32) * inv, axis=0, keepdims=True) * scale
    t_hi = jnp.sum(s_ref[1].astype(jnp.float32) * inv, axis=0, keepdims=True) * scale
    frac = jnp.sum(ind) * scale
    ge = (jnp.dot(t_lo, wb2l_ref[...], preferred_element_type=jnp.float32)
          + jnp.dot(t_hi, wb2h_ref[...], preferred_element_type=jnp.float32)
          + frac * bb2_ref[...])
    z = jnp.maximum(
        jnp.dot(st_ref[...], f1s_ref[...], preferred_element_type=jnp.float32)
        + jnp.dot(ge, f1g_ref[...], preferred_element_type=jnp.float32)
        + b1_ref[...], 0.0)
    z = jnp.maximum(
        jnp.dot(z, f2_ref[...], preferred_element_type=jnp.float32) + b2_ref[...], 0.0)
    mean_ref[...] = (jnp.dot(z, wm_ref[...], preferred_element_type=jnp.float32)
                     + bm_ref[...])
    ls_ref[...] = jnp.clip(
        jnp.dot(z, wl_ref[...], preferred_element_type=jnp.float32) + bl_ref[...],
        -20.0, 2.0)


def kernel(state, x, edge_index, g1_Wa, g1_ba, g1_Wb, g1_bb,
           g2_Wa, g2_ba, g2_Wb, g2_bb, fc1_W, fc1_b, fc2_W, fc2_b,
           mean_W, mean_b, ls_W, ls_b):
    n, f = x.shape
    hh = g1_Wa.shape[0]
    g = g2_Wa.shape[0]
    e = edge_index.shape[1]

    epw = e // _NS
    nch, _, _ = _edge_geom(epw)
    ei3 = edge_index.astype(jnp.int32).reshape(2, _NS, epw)
    pad = nch * _CHUNK - epw
    if pad:
        ei3 = jnp.pad(ei3, ((0, 0), (0, 0), (0, pad)))
    eidx = ei3.reshape(2, _NS, nch, _CHUNK)
    hh2 = hh // 2
    g2c = g // 2

    wi_t = g1_Wa[:, :f].T
    wj_t = g1_Wa[:, f:].T
    wb1_t = g1_Wb.T
    wa2i_t = g2_Wa[:, :hh].T
    wa2j_t = g2_Wa[:, hh:].T
    wb2_t = g2_Wb.T
    fc1_t, fc2_t = fc1_W.T, fc2_W.T
    mean_t, ls_t = mean_W.T, ls_W.T
    ba1 = g1_ba.reshape(1, hh)
    bb1 = g1_bb.reshape(1, hh)
    ba2 = g2_ba.reshape(1, g)
    bb2 = g2_bb.reshape(1, g)
    b1 = fc1_b.reshape(1, -1)
    b2 = fc2_b.reshape(1, -1)
    bm = mean_b.reshape(1, -1)
    bl = ls_b.reshape(1, -1)

    pshape = jax.ShapeDtypeStruct((n, hh2), jnp.bfloat16)
    pi_lo, pi_hi, pj_lo, pj_hi = pl.pallas_call(
        _proj_body,
        out_shape=[pshape, pshape, pshape, pshape],
    )(x, wi_t[:, :hh2], wi_t[:, hh2:], wj_t[:, :hh2], wj_t[:, hh2:],
      ba1[:, :hh2], ba1[:, hh2:])

    s1p, cntp = _make_sc_edge_pass(n, hh2, e, with_cnt=True)(
        pi_lo, pi_hi, pj_lo, pj_hi, eidx)

    qshape = jax.ShapeDtypeStruct((n, g2c), jnp.bfloat16)
    qi_lo, qi_hi, qj_lo, qj_hi = pl.pallas_call(
        _mid_body,
        out_shape=[qshape, qshape, qshape, qshape],
    )(s1p, cntp, wb1_t[:hh2], wb1_t[hh2:], bb1,
      wa2i_t[:, :g2c], wa2i_t[:, g2c:], wa2j_t[:, :g2c], wa2j_t[:, g2c:],
      ba2[:, :g2c], ba2[:, g2c:])

    (s2p,) = _make_sc_edge_pass(n, g2c, e, with_cnt=False)(
        qi_lo, qi_hi, qj_lo, qj_hi, eidx)

    mean, log_std = pl.pallas_call(
        functools.partial(_final_body, n),
        out_shape=[
            jax.ShapeDtypeStruct((1, mean_t.shape[1]), jnp.float32),
            jax.ShapeDtypeStruct((1, ls_t.shape[1]), jnp.float32),
        ],
    )(s2p, cntp, state, wb2_t[:g2c], wb2_t[g2c:], bb2,
      fc1_t[:f], fc1_t[f:], b1, fc2_t, b2, mean_t, bm, ls_t, bl)

    return (mean, log_std)
